# initial kernel scaffold (unmeasured)
import jax
import jax.numpy as jnp
from jax import lax
from jax.experimental import pallas as pl
from jax.experimental.pallas import tpu as pltpu

B, S, D = 2, 512, 2048
DC_HALF = 128
H, DH, DR = 16, 128, 32
SCALE = (DH + DR) ** -0.5


def _dot(a, b, contract=((1,), (0,))):
    return lax.dot_general(
        a, b, (contract, ((), ())), preferred_element_type=jnp.float32
    )


def _kv_body(x_ref, wdkv_ref, wuk_ref, wuv_ref, k_ref, v_ref,
             c_loc, c_rcv, wuk_rcv, wuv_rcv, send_sems, recv_sems):
    my_x = lax.axis_index("x")
    peer = (1 - my_x, lax.axis_index("y"), lax.axis_index("z"))

    barrier = pltpu.get_barrier_semaphore()
    pl.semaphore_signal(barrier, inc=1, device_id=peer,
                        device_id_type=pl.DeviceIdType.MESH)
    pl.semaphore_wait(barrier, 1)

    for b in range(B):
        c_loc[b, :, :] = _dot(x_ref[b], wdkv_ref[...])

    rdmas = []
    for i, (src, dst) in enumerate(
        [(c_loc, c_rcv), (wuk_ref, wuk_rcv), (wuv_ref, wuv_rcv)]
    ):
        rdma = pltpu.make_async_remote_copy(
            src_ref=src, dst_ref=dst,
            send_sem=send_sems.at[i], recv_sem=recv_sems.at[i],
            device_id=peer, device_id_type=pl.DeviceIdType.MESH,
        )
        rdma.start()
        rdmas.append(rdma)

    for b in range(B):
        k_ref[b, :, :] = _dot(c_loc[b], wuk_ref[...])
        v_ref[b, :, :] = _dot(c_loc[b], wuv_ref[...])

    for rdma in rdmas:
        rdma.wait()

    for b in range(B):
        k_ref[b, :, :] = k_ref[b] + _dot(c_rcv[b], wuk_rcv[...])
        v_ref[b, :, :] = v_ref[b] + _dot(c_rcv[b], wuv_rcv[...])


def _qproj_body(x_ref, wq_ref, wqr_ref, wkr_ref, q_ref, qr_ref, kr_ref):
    for b in range(B):
        xb = x_ref[b]
        q_ref[b, :, :] = _dot(xb, wq_ref[...])
        qr_ref[b, :, :] = _dot(xb, wqr_ref[...])
        kr_ref[b, :, :] = _dot(xb, wkr_ref[...])


def _attn_body(q_ref, k_ref, v_ref, qr_ref, kr_ref, o_ref):
    q, k, v = q_ref[0], k_ref[0], v_ref[0]
    qr, kr = qr_ref[0], kr_ref[0]
    s = (_dot(q, k, ((1,), (1,))) + _dot(qr, kr, ((1,), (1,)))) * SCALE
    m = jnp.max(s, axis=-1, keepdims=True)
    p = jnp.exp(s - m)
    p = p / jnp.sum(p, axis=-1, keepdims=True)
    o_ref[0] = _dot(p, v, ((1,), (0,)))


def _out_body(o_ref, wo_ref, y_ref):
    for b in range(B):
        y_ref[b, :, :] = _dot(o_ref[b], wo_ref[...])


def kernel(x, Wdkv, Wuk, Wuv, Wq, Wqr, Wkr, Wo):
    f32 = jnp.float32
    vmem = pl.BlockSpec(memory_space=pltpu.VMEM)

    k, v = pl.pallas_call(
        _kv_body,
        out_shape=[jax.ShapeDtypeStruct((B, S, D), f32)] * 2,
        in_specs=[vmem] * 4,
        out_specs=[vmem] * 2,
        scratch_shapes=[
            pltpu.VMEM((B, S, DC_HALF), f32),
            pltpu.VMEM((B, S, DC_HALF), f32),
            pltpu.VMEM((DC_HALF, D), f32),
            pltpu.VMEM((DC_HALF, D), f32),
            pltpu.SemaphoreType.DMA((3,)),
            pltpu.SemaphoreType.DMA((3,)),
        ],
        compiler_params=pltpu.CompilerParams(collective_id=0),
    )(x, Wdkv, Wuk, Wuv)

    q, qr, kr = pl.pallas_call(
        _qproj_body,
        out_shape=[
            jax.ShapeDtypeStruct((B, S, D), f32),
            jax.ShapeDtypeStruct((B, S, H * DR), f32),
            jax.ShapeDtypeStruct((B, S, DR), f32),
        ],
        in_specs=[vmem] * 4,
        out_specs=[vmem] * 3,
    )(x, Wq, Wqr, Wkr)

    o = pl.pallas_call(
        _attn_body,
        grid=(B, H),
        in_specs=[
            pl.BlockSpec((1, S, DH), lambda b, h: (b, 0, h)),
            pl.BlockSpec((1, S, DH), lambda b, h: (b, 0, h)),
            pl.BlockSpec((1, S, DH), lambda b, h: (b, 0, h)),
            pl.BlockSpec((1, S, DR), lambda b, h: (b, 0, h)),
            pl.BlockSpec((1, S, DR), lambda b, h: (b, 0, 0)),
        ],
        out_specs=pl.BlockSpec((1, S, DH), lambda b, h: (b, 0, h)),
        out_shape=jax.ShapeDtypeStruct((B, S, D), f32),
    )(q, k, v, qr, kr)

    return pl.pallas_call(
        _out_body,
        out_shape=jax.ShapeDtypeStruct((B, S, D), f32),
        in_specs=[vmem] * 2,
        out_specs=vmem,
    )(o, Wo)


# baseline (device time: 131312 ns/iter reference)
import jax
import jax.numpy as jnp
from jax import lax
from jax.experimental import pallas as pl
from jax.experimental.pallas import tpu as pltpu

B, S, D = 2, 512, 2048
DC_HALF = 128
H, DH, DR = 16, 128, 32
SCALE = (DH + DR) ** -0.5


def _dot(a, b, contract=((1,), (0,))):
    return lax.dot_general(
        a, b, (contract, ((), ())), preferred_element_type=jnp.float32
    )


def _kv_body(x_ref, wdkv_ref, wuk_ref, wuv_ref, k_ref, v_ref,
             c_loc, c_rcv, wuk_rcv, wuv_rcv, send_sems, recv_sems):
    my_x = lax.axis_index("x")
    peer = (1 - my_x, lax.axis_index("y"), lax.axis_index("z"))

    barrier = pltpu.get_barrier_semaphore()
    pl.semaphore_signal(barrier, inc=1, device_id=peer,
                        device_id_type=pl.DeviceIdType.MESH)
    pl.semaphore_wait(barrier, 1)

    for b in range(B):
        c_loc[b, :, :] = _dot(x_ref[b], wdkv_ref[...])

    rdmas = []
    for i, (src, dst) in enumerate(
        [(c_loc, c_rcv), (wuk_ref, wuk_rcv), (wuv_ref, wuv_rcv)]
    ):
        rdma = pltpu.make_async_remote_copy(
            src_ref=src, dst_ref=dst,
            send_sem=send_sems.at[i], recv_sem=recv_sems.at[i],
            device_id=peer, device_id_type=pl.DeviceIdType.MESH,
        )
        rdma.start()
        rdmas.append(rdma)

    for b in range(B):
        k_ref[b, :, :] = _dot(c_loc[b], wuk_ref[...])
        v_ref[b, :, :] = _dot(c_loc[b], wuv_ref[...])

    for rdma in rdmas:
        rdma.wait()

    for b in range(B):
        k_ref[b, :, :] = k_ref[b] + _dot(c_rcv[b], wuk_rcv[...])
        v_ref[b, :, :] = v_ref[b] + _dot(c_rcv[b], wuv_rcv[...])


def _qproj_body(x_ref, wq_ref, wqr_ref, wkr_ref, q_ref, qr_ref, kr_ref):
    for b in range(B):
        xb = x_ref[b]
        q_ref[b, :, :] = _dot(xb, wq_ref[...])
        qr_full = _dot(xb, wqr_ref[...])
        for h in range(H):
            qr_ref[b, h, :, :] = qr_full[:, h * DR:(h + 1) * DR]
        kr_ref[b, :, :] = _dot(xb, wkr_ref[...])


def _attn_body(q_ref, k_ref, v_ref, qr_ref, kr_ref, o_ref):
    q, k, v = q_ref[0], k_ref[0], v_ref[0]
    qr, kr = qr_ref[0, 0], kr_ref[0]
    s = (_dot(q, k, ((1,), (1,))) + _dot(qr, kr, ((1,), (1,)))) * SCALE
    m = jnp.max(s, axis=-1, keepdims=True)
    p = jnp.exp(s - m)
    p = p / jnp.sum(p, axis=-1, keepdims=True)
    o_ref[0] = _dot(p, v, ((1,), (0,)))


def _out_body(o_ref, wo_ref, y_ref):
    for b in range(B):
        y_ref[b, :, :] = _dot(o_ref[b], wo_ref[...])


def kernel(x, Wdkv, Wuk, Wuv, Wq, Wqr, Wkr, Wo):
    f32 = jnp.float32
    vmem = pl.BlockSpec(memory_space=pltpu.VMEM)

    k, v = pl.pallas_call(
        _kv_body,
        out_shape=[jax.ShapeDtypeStruct((B, S, D), f32)] * 2,
        in_specs=[vmem] * 4,
        out_specs=[vmem] * 2,
        scratch_shapes=[
            pltpu.VMEM((B, S, DC_HALF), f32),
            pltpu.VMEM((B, S, DC_HALF), f32),
            pltpu.VMEM((DC_HALF, D), f32),
            pltpu.VMEM((DC_HALF, D), f32),
            pltpu.SemaphoreType.DMA((3,)),
            pltpu.SemaphoreType.DMA((3,)),
        ],
        compiler_params=pltpu.CompilerParams(collective_id=0),
    )(x, Wdkv, Wuk, Wuv)

    q, qr, kr = pl.pallas_call(
        _qproj_body,
        out_shape=[
            jax.ShapeDtypeStruct((B, S, D), f32),
            jax.ShapeDtypeStruct((B, H, S, DR), f32),
            jax.ShapeDtypeStruct((B, S, DR), f32),
        ],
        in_specs=[vmem] * 4,
        out_specs=[vmem] * 3,
    )(x, Wq, Wqr, Wkr)

    o = pl.pallas_call(
        _attn_body,
        grid=(B, H),
        in_specs=[
            pl.BlockSpec((1, S, DH), lambda b, h: (b, 0, h)),
            pl.BlockSpec((1, S, DH), lambda b, h: (b, 0, h)),
            pl.BlockSpec((1, S, DH), lambda b, h: (b, 0, h)),
            pl.BlockSpec((1, 1, S, DR), lambda b, h: (b, h, 0, 0)),
            pl.BlockSpec((1, S, DR), lambda b, h: (b, 0, 0)),
        ],
        out_specs=pl.BlockSpec((1, S, DH), lambda b, h: (b, 0, h)),
        out_shape=jax.ShapeDtypeStruct((B, S, D), f32),
    )(q, k, v, qr, kr)

    return pl.pallas_call(
        _out_body,
        out_shape=jax.ShapeDtypeStruct((B, S, D), f32),
        in_specs=[vmem] * 2,
        out_specs=vmem,
    )(o, Wo)


# device time: 108116 ns/iter; 1.2145x vs baseline; 1.2145x over previous
import jax
import jax.numpy as jnp
from jax import lax
from jax.experimental import pallas as pl
from jax.experimental.pallas import tpu as pltpu

B, S, D = 2, 512, 2048
DC_HALF = 128
H, DH, DR = 16, 128, 32
SCALE = (DH + DR) ** -0.5


def _dot(a, b, contract=((1,), (0,))):
    return lax.dot_general(
        a, b, (contract, ((), ())), preferred_element_type=jnp.float32
    )


def _kvq_body(x_ref, wdkv_ref, wuk_ref, wuv_ref, wq_ref, wqr_ref, wkr_ref,
              k_ref, v_ref, q_ref, qr_ref, kr_ref,
              c_loc, c_rcv, wuk_rcv, wuv_rcv, send_sems, recv_sems):
    my_x = lax.axis_index("x")
    peer = (1 - my_x, lax.axis_index("y"), lax.axis_index("z"))

    barrier = pltpu.get_barrier_semaphore()
    pl.semaphore_signal(barrier, inc=1, device_id=peer,
                        device_id_type=pl.DeviceIdType.MESH)
    pl.semaphore_wait(barrier, 1)

    for b in range(B):
        c_loc[b, :, :] = _dot(x_ref[b], wdkv_ref[...])

    rdmas = []
    for i, (src, dst) in enumerate(
        [(c_loc, c_rcv), (wuk_ref, wuk_rcv), (wuv_ref, wuv_rcv)]
    ):
        rdma = pltpu.make_async_remote_copy(
            src_ref=src, dst_ref=dst,
            send_sem=send_sems.at[i], recv_sem=recv_sems.at[i],
            device_id=peer, device_id_type=pl.DeviceIdType.MESH,
        )
        rdma.start()
        rdmas.append(rdma)

    for b in range(B):
        xb = x_ref[b]
        q_ref[b, :, :] = _dot(xb, wq_ref[...])
        qr_ref[b, :, :] = _dot(xb, wqr_ref[...])
        kr_ref[b, :, :] = _dot(xb, wkr_ref[...])
        k_ref[b, :, :] = _dot(c_loc[b], wuk_ref[...])
        v_ref[b, :, :] = _dot(c_loc[b], wuv_ref[...])

    for rdma in rdmas:
        rdma.wait()

    for b in range(B):
        k_ref[b, :, :] = k_ref[b] + _dot(c_rcv[b], wuk_rcv[...])
        v_ref[b, :, :] = v_ref[b] + _dot(c_rcv[b], wuv_rcv[...])


def _attn_body(q_ref, k_ref, v_ref, qr_ref, kr_ref, o_ref):
    kr = kr_ref[0]
    for h in range(H):
        q = q_ref[0, :, h * DH:(h + 1) * DH]
        k = k_ref[0, :, h * DH:(h + 1) * DH]
        v = v_ref[0, :, h * DH:(h + 1) * DH]
        qr = qr_ref[0, :, h * DR:(h + 1) * DR]
        s = (_dot(q, k, ((1,), (1,))) + _dot(qr, kr, ((1,), (1,)))) * SCALE
        m = jnp.max(s, axis=-1, keepdims=True)
        p = jnp.exp(s - m)
        p = p / jnp.sum(p, axis=-1, keepdims=True)
        o_ref[0, :, h * DH:(h + 1) * DH] = _dot(p, v, ((1,), (0,)))


def _out_body(o_ref, wo_ref, y_ref):
    for b in range(B):
        y_ref[b, :, :] = _dot(o_ref[b], wo_ref[...])


def kernel(x, Wdkv, Wuk, Wuv, Wq, Wqr, Wkr, Wo):
    f32 = jnp.float32
    vmem = pl.BlockSpec(memory_space=pltpu.VMEM)

    k, v, q, qr, kr = pl.pallas_call(
        _kvq_body,
        out_shape=[
            jax.ShapeDtypeStruct((B, S, D), f32),
            jax.ShapeDtypeStruct((B, S, D), f32),
            jax.ShapeDtypeStruct((B, S, D), f32),
            jax.ShapeDtypeStruct((B, S, H * DR), f32),
            jax.ShapeDtypeStruct((B, S, DR), f32),
        ],
        in_specs=[vmem] * 7,
        out_specs=[vmem] * 5,
        scratch_shapes=[
            pltpu.VMEM((B, S, DC_HALF), f32),
            pltpu.VMEM((B, S, DC_HALF), f32),
            pltpu.VMEM((DC_HALF, D), f32),
            pltpu.VMEM((DC_HALF, D), f32),
            pltpu.SemaphoreType.DMA((3,)),
            pltpu.SemaphoreType.DMA((3,)),
        ],
        compiler_params=pltpu.CompilerParams(
            collective_id=0, vmem_limit_bytes=100 * 1024 * 1024
        ),
    )(x, Wdkv, Wuk, Wuv, Wq, Wqr, Wkr)

    o = pl.pallas_call(
        _attn_body,
        grid=(B,),
        in_specs=[
            pl.BlockSpec((1, S, D), lambda b: (b, 0, 0)),
            pl.BlockSpec((1, S, D), lambda b: (b, 0, 0)),
            pl.BlockSpec((1, S, D), lambda b: (b, 0, 0)),
            pl.BlockSpec((1, S, H * DR), lambda b: (b, 0, 0)),
            pl.BlockSpec((1, S, DR), lambda b: (b, 0, 0)),
        ],
        out_specs=pl.BlockSpec((1, S, D), lambda b: (b, 0, 0)),
        out_shape=jax.ShapeDtypeStruct((B, S, D), f32),
    )(q, k, v, qr, kr)

    return pl.pallas_call(
        _out_body,
        out_shape=jax.ShapeDtypeStruct((B, S, D), f32),
        in_specs=[vmem] * 2,
        out_specs=vmem,
    )(o, Wo)
